# Initial kernel scaffold; baseline (speedup 1.0000x reference)
import jax
import jax.numpy as jnp
from jax.experimental import pallas as pl


def kernel(x, size, edge_index, edge_attr, W0, a_src0, a_dst0, We0, W1, a_src1, a_dst1, We1):
    raise NotImplementedError("write your pallas kernel here")


# speedup vs baseline: 51.5048x; 51.5048x over previous
import functools

import jax
import jax.numpy as jnp
from jax import lax
from jax.experimental import pallas as pl
from jax.experimental.pallas import tpu as pltpu
from jax.experimental.pallas import tpu_sc as plsc

H = 8
DH = 16
D = 128
HP = 16
NC = 2
NS = 16
NW = NC * NS
C = 80


def _pre_body(x_ref, w_ref, as_ref, ad_ref, h_ref, als_ref, ald_ref):
    h = jnp.dot(x_ref[...], w_ref[...], preferred_element_type=jnp.float32)
    h_ref[...] = h
    als_ref[...] = jnp.dot(h, as_ref[...], preferred_element_type=jnp.float32)
    ald_ref[...] = jnp.dot(h, ad_ref[...], preferred_element_type=jnp.float32)


def _tc_pre(x, w, a_s, a_d, bn):
    n = x.shape[0]
    grid = n // bn
    return pl.pallas_call(
        _pre_body,
        grid=(grid,),
        in_specs=[
            pl.BlockSpec((bn, D), lambda i: (i, 0)),
            pl.BlockSpec((D, D), lambda i: (0, 0)),
            pl.BlockSpec((D, HP), lambda i: (0, 0)),
            pl.BlockSpec((D, HP), lambda i: (0, 0)),
        ],
        out_specs=[
            pl.BlockSpec((bn, D), lambda i: (i, 0)),
            pl.BlockSpec((bn, HP), lambda i: (i, 0)),
            pl.BlockSpec((bn, HP), lambda i: (i, 0)),
        ],
        out_shape=[
            jax.ShapeDtypeStruct((n, D), jnp.float32),
            jax.ShapeDtypeStruct((n, HP), jnp.float32),
            jax.ShapeDtypeStruct((n, HP), jnp.float32),
        ],
    )(x, w, a_s, a_d)


def _e_body(ea_ref, w0_ref, w1_ref, e0_ref, e1_ref):
    ea = ea_ref[...]
    e0_ref[...] = jnp.dot(ea, w0_ref[...], preferred_element_type=jnp.float32)
    e1_ref[...] = jnp.dot(ea, w1_ref[...], preferred_element_type=jnp.float32)


def _tc_edge_proj(edge_attr, we0p, we1p, be):
    e_cnt, de = edge_attr.shape
    grid = e_cnt // be
    return pl.pallas_call(
        _e_body,
        grid=(grid,),
        in_specs=[
            pl.BlockSpec((be, de), lambda i: (i, 0)),
            pl.BlockSpec((de, HP), lambda i: (0, 0)),
            pl.BlockSpec((de, HP), lambda i: (0, 0)),
        ],
        out_specs=[
            pl.BlockSpec((be, HP), lambda i: (i, 0)),
            pl.BlockSpec((be, HP), lambda i: (i, 0)),
        ],
        out_shape=[
            jax.ShapeDtypeStruct((e_cnt, HP), jnp.float32),
            jax.ShapeDtypeStruct((e_cnt, HP), jnp.float32),
        ],
    )(edge_attr, we0p, we1p)


def _finish(num_ref, den_ref, exp_ref):
    numt = num_ref[0] + num_ref[1]
    dent = den_ref[0] + den_ref[1]
    dexp = jnp.dot(dent, exp_ref[...], preferred_element_type=jnp.float32)
    o = numt / (dexp + 1e-16)
    return jnp.where(o > 0, o, jnp.exp(o) - 1.0)


def _mid_body(num_ref, den_ref, exp_ref, w_ref, as_ref, ad_ref,
              h_ref, als_ref, ald_ref):
    x1 = _finish(num_ref, den_ref, exp_ref)
    h = jnp.dot(x1, w_ref[...], preferred_element_type=jnp.float32)
    h_ref[...] = h
    als_ref[...] = jnp.dot(h, as_ref[...], preferred_element_type=jnp.float32)
    ald_ref[...] = jnp.dot(h, ad_ref[...], preferred_element_type=jnp.float32)


def _tc_mid(num, den, expm, w, a_s, a_d, bn, n):
    grid = n // bn
    return pl.pallas_call(
        _mid_body,
        grid=(grid,),
        in_specs=[
            pl.BlockSpec((NC, bn, D), lambda i: (0, i, 0)),
            pl.BlockSpec((NC, bn, HP), lambda i: (0, i, 0)),
            pl.BlockSpec((HP, D), lambda i: (0, 0)),
            pl.BlockSpec((D, D), lambda i: (0, 0)),
            pl.BlockSpec((D, HP), lambda i: (0, 0)),
            pl.BlockSpec((D, HP), lambda i: (0, 0)),
        ],
        out_specs=[
            pl.BlockSpec((bn, D), lambda i: (i, 0)),
            pl.BlockSpec((bn, HP), lambda i: (i, 0)),
            pl.BlockSpec((bn, HP), lambda i: (i, 0)),
        ],
        out_shape=[
            jax.ShapeDtypeStruct((n, D), jnp.float32),
            jax.ShapeDtypeStruct((n, HP), jnp.float32),
            jax.ShapeDtypeStruct((n, HP), jnp.float32),
        ],
    )(num, den, expm, w, a_s, a_d)


def _post_body(num_ref, den_ref, exp_ref, out_ref):
    out_ref[...] = _finish(num_ref, den_ref, exp_ref)


def _tc_post(num, den, expm, bn, n):
    grid = n // bn
    return pl.pallas_call(
        _post_body,
        grid=(grid,),
        in_specs=[
            pl.BlockSpec((NC, bn, D), lambda i: (0, i, 0)),
            pl.BlockSpec((NC, bn, HP), lambda i: (0, i, 0)),
            pl.BlockSpec((HP, D), lambda i: (0, 0)),
        ],
        out_specs=pl.BlockSpec((bn, D), lambda i: (i, 0)),
        out_shape=jax.ShapeDtypeStruct((n, D), jnp.float32),
    )(num, den, expm)


def _sc_edges(src, dst, h, als, ald, e, zd, zh):
    n_pad = zd.shape[0]
    e_cnt = src.shape[0]
    ew = e_cnt // NW
    nchunk = ew // C
    rt = n_pad // NS

    mesh = plsc.VectorSubcoreMesh(core_axis_name="c", subcore_axis_name="s")

    @functools.partial(
        pl.kernel,
        mesh=mesh,
        compiler_params=pltpu.CompilerParams(use_tc_tiling_on_sc=False),
        out_type=[
            jax.ShapeDtypeStruct((NC, n_pad, D), jnp.float32),
            jax.ShapeDtypeStruct((NC, n_pad, HP), jnp.float32),
        ],
        scratch_types=[
            pltpu.VMEM((C,), jnp.int32),
            pltpu.VMEM((C,), jnp.int32),
            pltpu.VMEM((C, HP), jnp.float32),
            pltpu.VMEM((C, HP), jnp.float32),
            pltpu.VMEM((C, HP), jnp.float32),
            pltpu.VMEM((C, HP), jnp.float32),
            pltpu.VMEM((C, D), jnp.float32),
            pltpu.VMEM_SHARED((n_pad, D), jnp.float32),
            pltpu.VMEM_SHARED((n_pad, HP), jnp.float32),
            pltpu.SemaphoreType.DMA,
            pltpu.SemaphoreType.DMA,
            pltpu.SemaphoreType.DMA,
            pltpu.SemaphoreType.DMA,
        ],
    )
    def sc_kernel(src_hbm, dst_hbm, h_hbm, als_hbm, ald_hbm, e_hbm,
                  zd_hbm, zh_hbm, num_out, den_out,
                  src_v, dst_v, als_v, ald_v, e_v, ex_v, h_v,
                  num_sh, den_sh, sem0, sem1, sem2, sem3):
        cid = lax.axis_index("c")
        sid = lax.axis_index("s")
        wid = sid * NC + cid
        base0 = wid * ew
        zrow = sid * rt

        pltpu.sync_copy(zd_hbm.at[pl.ds(zrow, rt)], num_sh.at[pl.ds(zrow, rt)])
        pltpu.sync_copy(zh_hbm.at[pl.ds(zrow, rt)], den_sh.at[pl.ds(zrow, rt)])
        plsc.subcore_barrier()

        def chunk_body(j, carry):
            base = base0 + j * C
            pltpu.sync_copy(src_hbm.at[pl.ds(base, C)], src_v)
            pltpu.sync_copy(dst_hbm.at[pl.ds(base, C)], dst_v)
            cp0 = pltpu.async_copy(als_hbm.at[src_v], als_v, sem0)
            cp1 = pltpu.async_copy(ald_hbm.at[dst_v], ald_v, sem1)
            cp2 = pltpu.async_copy(e_hbm.at[pl.ds(base, C)], e_v, sem2)
            cp3 = pltpu.async_copy(h_hbm.at[src_v], h_v, sem3)
            cp0.wait()
            cp1.wait()
            cp2.wait()
            cp3.wait()

            def edge_body(c, carry2):
                lg = als_v[c] + ald_v[c] + e_v[c]
                lg = jnp.where(lg >= 0.0, lg, 0.2 * lg)
                exv = jnp.exp(lg)
                ex_v[c] = exv
                for k in range(H):
                    hk = h_v[c, pl.ds(k * DH, DH)]
                    mk = lax.broadcast_in_dim(exv[k], (DH,), ())
                    h_v[c, pl.ds(k * DH, DH)] = hk * mk
                return carry2

            lax.fori_loop(0, C, edge_body, 0, unroll=2)
            pltpu.sync_copy(ex_v, den_sh.at[dst_v], add=True)
            pltpu.sync_copy(h_v, num_sh.at[dst_v], add=True)
            return carry

        lax.fori_loop(0, nchunk, chunk_body, 0)
        plsc.subcore_barrier()
        pltpu.sync_copy(num_sh.at[pl.ds(zrow, rt)],
                        num_out.at[cid, pl.ds(zrow, rt)])
        pltpu.sync_copy(den_sh.at[pl.ds(zrow, rt)],
                        den_out.at[cid, pl.ds(zrow, rt)])

    return sc_kernel(src, dst, h, als, ald, e, zd, zh)


def _attn_mat(a):
    rows = jnp.arange(D)
    mask = (rows[:, None] // DH) == jnp.arange(HP)[None, :]
    return mask.astype(jnp.float32) * a.reshape(D)[:, None]


def kernel(x, size, edge_index, edge_attr, W0, a_src0, a_dst0, We0,
           W1, a_src1, a_dst1, We1):
    del size
    n = x.shape[0]
    src = edge_index[0]
    dst = edge_index[1]

    as0 = _attn_mat(a_src0)
    ad0 = _attn_mat(a_dst0)
    as1 = _attn_mat(a_src1)
    ad1 = _attn_mat(a_dst1)
    rows = jnp.arange(D)
    expm = ((rows[None, :] // DH) == jnp.arange(HP)[:, None]).astype(jnp.float32)
    zpad = jnp.zeros((We0.shape[0], HP - H), jnp.float32)
    we0p = jnp.concatenate([We0, zpad], axis=1)
    we1p = jnp.concatenate([We1, zpad], axis=1)
    n_pad = -(-n // (NS * 8)) * (NS * 8)
    zd = jnp.zeros((n_pad, D), jnp.float32)
    zh = jnp.zeros((n_pad, HP), jnp.float32)

    bn = 2000
    h0, als0, ald0 = _tc_pre(x, W0, as0, ad0, bn)
    e0, e1 = _tc_edge_proj(edge_attr, we0p, we1p, 8000)
    num0, den0 = _sc_edges(src, dst, h0, als0, ald0, e0, zd, zh)
    h1, als1, ald1 = _tc_mid(num0, den0, expm, W1, as1, ad1, bn, n)
    num1, den1 = _sc_edges(src, dst, h1, als1, ald1, e1, zd, zh)
    return _tc_post(num1, den1, expm, bn, n)


# speedup vs baseline: 68.6381x; 1.3327x over previous
import functools

import jax
import jax.numpy as jnp
from jax import lax
from jax.experimental import pallas as pl
from jax.experimental.pallas import tpu as pltpu
from jax.experimental.pallas import tpu_sc as plsc

H = 8
DH = 16
D = 128
HP = 16
NC = 2
NS = 16
NW = NC * NS
C = 80


def _pre_body(x_ref, w_ref, as_ref, ad_ref, h_ref, als_ref, ald_ref):
    h = jnp.dot(x_ref[...], w_ref[...], preferred_element_type=jnp.float32)
    h_ref[...] = h
    als_ref[...] = jnp.dot(h, as_ref[...], preferred_element_type=jnp.float32)
    ald_ref[...] = jnp.dot(h, ad_ref[...], preferred_element_type=jnp.float32)


def _tc_pre(x, w, a_s, a_d, bn):
    n = x.shape[0]
    grid = n // bn
    return pl.pallas_call(
        _pre_body,
        grid=(grid,),
        in_specs=[
            pl.BlockSpec((bn, D), lambda i: (i, 0)),
            pl.BlockSpec((D, D), lambda i: (0, 0)),
            pl.BlockSpec((D, HP), lambda i: (0, 0)),
            pl.BlockSpec((D, HP), lambda i: (0, 0)),
        ],
        out_specs=[
            pl.BlockSpec((bn, D), lambda i: (i, 0)),
            pl.BlockSpec((bn, HP), lambda i: (i, 0)),
            pl.BlockSpec((bn, HP), lambda i: (i, 0)),
        ],
        out_shape=[
            jax.ShapeDtypeStruct((n, D), jnp.float32),
            jax.ShapeDtypeStruct((n, HP), jnp.float32),
            jax.ShapeDtypeStruct((n, HP), jnp.float32),
        ],
    )(x, w, a_s, a_d)


def _e_body(ea_ref, w0_ref, w1_ref, e0_ref, e1_ref):
    ea = ea_ref[...]
    e0_ref[...] = jnp.dot(ea, w0_ref[...], preferred_element_type=jnp.float32)
    e1_ref[...] = jnp.dot(ea, w1_ref[...], preferred_element_type=jnp.float32)


def _tc_edge_proj(edge_attr, we0p, we1p, be):
    e_cnt, de = edge_attr.shape
    grid = e_cnt // be
    return pl.pallas_call(
        _e_body,
        grid=(grid,),
        in_specs=[
            pl.BlockSpec((be, de), lambda i: (i, 0)),
            pl.BlockSpec((de, HP), lambda i: (0, 0)),
            pl.BlockSpec((de, HP), lambda i: (0, 0)),
        ],
        out_specs=[
            pl.BlockSpec((be, HP), lambda i: (i, 0)),
            pl.BlockSpec((be, HP), lambda i: (i, 0)),
        ],
        out_shape=[
            jax.ShapeDtypeStruct((e_cnt, HP), jnp.float32),
            jax.ShapeDtypeStruct((e_cnt, HP), jnp.float32),
        ],
    )(edge_attr, we0p, we1p)


def _finish(num_ref, den_ref, exp_ref):
    numt = num_ref[0] + num_ref[1]
    dent = den_ref[0] + den_ref[1]
    dexp = jnp.dot(dent, exp_ref[...], preferred_element_type=jnp.float32)
    o = numt / (dexp + 1e-16)
    return jnp.where(o > 0, o, jnp.exp(o) - 1.0)


def _mid_body(num_ref, den_ref, exp_ref, w_ref, as_ref, ad_ref,
              h_ref, als_ref, ald_ref):
    x1 = _finish(num_ref, den_ref, exp_ref)
    h = jnp.dot(x1, w_ref[...], preferred_element_type=jnp.float32)
    h_ref[...] = h
    als_ref[...] = jnp.dot(h, as_ref[...], preferred_element_type=jnp.float32)
    ald_ref[...] = jnp.dot(h, ad_ref[...], preferred_element_type=jnp.float32)


def _tc_mid(num, den, expm, w, a_s, a_d, bn, n):
    grid = n // bn
    return pl.pallas_call(
        _mid_body,
        grid=(grid,),
        in_specs=[
            pl.BlockSpec((NC, bn, D), lambda i: (0, i, 0)),
            pl.BlockSpec((NC, bn, HP), lambda i: (0, i, 0)),
            pl.BlockSpec((HP, D), lambda i: (0, 0)),
            pl.BlockSpec((D, D), lambda i: (0, 0)),
            pl.BlockSpec((D, HP), lambda i: (0, 0)),
            pl.BlockSpec((D, HP), lambda i: (0, 0)),
        ],
        out_specs=[
            pl.BlockSpec((bn, D), lambda i: (i, 0)),
            pl.BlockSpec((bn, HP), lambda i: (i, 0)),
            pl.BlockSpec((bn, HP), lambda i: (i, 0)),
        ],
        out_shape=[
            jax.ShapeDtypeStruct((n, D), jnp.float32),
            jax.ShapeDtypeStruct((n, HP), jnp.float32),
            jax.ShapeDtypeStruct((n, HP), jnp.float32),
        ],
    )(num, den, expm, w, a_s, a_d)


def _post_body(num_ref, den_ref, exp_ref, out_ref):
    out_ref[...] = _finish(num_ref, den_ref, exp_ref)


def _tc_post(num, den, expm, bn, n):
    grid = n // bn
    return pl.pallas_call(
        _post_body,
        grid=(grid,),
        in_specs=[
            pl.BlockSpec((NC, bn, D), lambda i: (0, i, 0)),
            pl.BlockSpec((NC, bn, HP), lambda i: (0, i, 0)),
            pl.BlockSpec((HP, D), lambda i: (0, 0)),
        ],
        out_specs=pl.BlockSpec((bn, D), lambda i: (i, 0)),
        out_shape=jax.ShapeDtypeStruct((n, D), jnp.float32),
    )(num, den, expm)


def _sc_edges(src, dst, h, als, ald, e, zd, zh):
    n = zd.shape[0]
    e_cnt = src.shape[0]
    ew = e_cnt // NW
    nchunk = ew // C
    rt = (n // NS) // 8 * 8
    rem = n - NS * rt

    mesh = plsc.VectorSubcoreMesh(core_axis_name="c", subcore_axis_name="s")

    @functools.partial(
        pl.kernel,
        mesh=mesh,
        compiler_params=pltpu.CompilerParams(use_tc_tiling_on_sc=False),
        out_type=[
            jax.ShapeDtypeStruct((NC, n, D), jnp.float32),
            jax.ShapeDtypeStruct((NC, n, HP), jnp.float32),
        ],
        scratch_types=[
            [pltpu.VMEM((C,), jnp.int32)] * 2,
            [pltpu.VMEM((C,), jnp.int32)] * 2,
            [pltpu.VMEM((C, HP), jnp.float32)] * 2,
            [pltpu.VMEM((C, HP), jnp.float32)] * 2,
            [pltpu.VMEM((C, HP), jnp.float32)] * 2,
            [pltpu.VMEM((C, HP), jnp.float32)] * 2,
            [pltpu.VMEM((C, D), jnp.float32)] * 2,
            pltpu.VMEM_SHARED((n, D), jnp.float32),
            pltpu.VMEM_SHARED((n, HP), jnp.float32),
            [pltpu.SemaphoreType.DMA] * 2,
            [pltpu.SemaphoreType.DMA] * 2,
            [pltpu.SemaphoreType.DMA] * 2,
        ],
    )
    def sc_kernel(src_hbm, dst_hbm, h_hbm, als_hbm, ald_hbm, e_hbm,
                  zd_hbm, zh_hbm, num_out, den_out,
                  src_v, dst_v, als_v, ald_v, e_v, ex_v, h_v,
                  num_sh, den_sh, gsem, ssem, isem):
        cid = lax.axis_index("c")
        sid = lax.axis_index("s")
        wid = sid * NC + cid
        base0 = wid * ew
        zrow = sid * rt

        def issue_idx(j, p):
            pltpu.async_copy(src_hbm.at[pl.ds(base0 + j * C, C)], src_v[p],
                             isem[p])
            pltpu.async_copy(dst_hbm.at[pl.ds(base0 + j * C, C)], dst_v[p],
                             isem[p])

        def wait_idx(p):
            pltpu.make_async_copy(src_hbm.at[pl.ds(base0, C)], src_v[p],
                                  isem[p]).wait()
            pltpu.make_async_copy(dst_hbm.at[pl.ds(base0, C)], dst_v[p],
                                  isem[p]).wait()

        issue_idx(0, 0)
        issue_idx(1, 1)
        pltpu.sync_copy(zd_hbm.at[pl.ds(zrow, rt)], num_sh.at[pl.ds(zrow, rt)])
        pltpu.sync_copy(zh_hbm.at[pl.ds(zrow, rt)], den_sh.at[pl.ds(zrow, rt)])

        @pl.when(sid == NS - 1)
        def _zero_tail():
            tr = NS * rt
            pltpu.sync_copy(zd_hbm.at[pl.ds(tr, rem)],
                            num_sh.at[pl.ds(tr, rem)])
            pltpu.sync_copy(zh_hbm.at[pl.ds(tr, rem)],
                            den_sh.at[pl.ds(tr, rem)])

        plsc.subcore_barrier()

        def issue_gathers(j, p):
            pltpu.async_copy(als_hbm.at[src_v[p]], als_v[p], gsem[p])
            pltpu.async_copy(ald_hbm.at[dst_v[p]], ald_v[p], gsem[p])
            pltpu.async_copy(e_hbm.at[pl.ds(base0 + j * C, C)], e_v[p],
                             gsem[p])
            pltpu.async_copy(h_hbm.at[src_v[p]], h_v[p], gsem[p])

        def wait_gathers(p):
            pltpu.make_async_copy(als_hbm.at[src_v[p]], als_v[p],
                                  gsem[p]).wait()
            pltpu.make_async_copy(ald_hbm.at[dst_v[p]], ald_v[p],
                                  gsem[p]).wait()
            pltpu.make_async_copy(e_hbm.at[pl.ds(base0, C)], e_v[p],
                                  gsem[p]).wait()
            pltpu.make_async_copy(h_hbm.at[src_v[p]], h_v[p],
                                  gsem[p]).wait()

        def compute(p):
            def edge_body(c, carry2):
                lg = als_v[p][c] + ald_v[p][c] + e_v[p][c]
                lg = jnp.where(lg >= 0.0, lg, 0.2 * lg)
                exv = jnp.exp(lg)
                ex_v[p][c] = exv
                for k in range(H):
                    hk = h_v[p][c, pl.ds(k * DH, DH)]
                    mk = lax.broadcast_in_dim(exv[k], (DH,), ())
                    h_v[p][c, pl.ds(k * DH, DH)] = hk * mk
                return carry2

            lax.fori_loop(0, C, edge_body, 0, unroll=2)

        def issue_scatters(p):
            pltpu.async_copy(ex_v[p], den_sh.at[dst_v[p]], ssem[p], add=True)
            pltpu.async_copy(h_v[p], num_sh.at[dst_v[p]], ssem[p], add=True)

        def wait_scatters(p):
            pltpu.make_async_copy(ex_v[p], den_sh.at[dst_v[p]],
                                  ssem[p]).wait()
            pltpu.make_async_copy(h_v[p], num_sh.at[dst_v[p]],
                                  ssem[p]).wait()

        wait_idx(0)
        issue_gathers(0, 0)
        wait_idx(1)
        issue_gathers(1, 1)
        wait_gathers(0)
        compute(0)
        issue_scatters(0)
        wait_scatters(0)
        issue_idx(2, 0)

        def half(j, p, q):
            wait_idx(q)
            issue_gathers(j + 1, q)
            wait_gathers(p)
            compute(p)
            issue_scatters(p)
            wait_scatters(p)
            issue_idx(j + 2, p)

        def loop_body(jj, carry):
            j0 = 2 * jj + 1
            half(j0, 1, 0)
            half(j0 + 1, 0, 1)
            return carry

        npair = (nchunk - 3) // 2
        lax.fori_loop(0, npair, loop_body, 0)

        wait_idx(0)
        issue_gathers(nchunk - 1, 0)
        wait_gathers(1)
        compute(1)
        issue_scatters(1)
        wait_scatters(1)
        wait_gathers(0)
        compute(0)
        issue_scatters(0)
        wait_scatters(0)

        plsc.subcore_barrier()
        pltpu.sync_copy(num_sh.at[pl.ds(zrow, rt)],
                        num_out.at[cid, pl.ds(zrow, rt)])
        pltpu.sync_copy(den_sh.at[pl.ds(zrow, rt)],
                        den_out.at[cid, pl.ds(zrow, rt)])

        @pl.when(sid == NS - 1)
        def _write_tail():
            tr = NS * rt
            pltpu.sync_copy(num_sh.at[pl.ds(tr, rem)],
                            num_out.at[cid, pl.ds(tr, rem)])
            pltpu.sync_copy(den_sh.at[pl.ds(tr, rem)],
                            den_out.at[cid, pl.ds(tr, rem)])

    return sc_kernel(src, dst, h, als, ald, e, zd, zh)


def _attn_mat(a):
    rows = jnp.arange(D)
    mask = (rows[:, None] // DH) == jnp.arange(HP)[None, :]
    return mask.astype(jnp.float32) * a.reshape(D)[:, None]


def kernel(x, size, edge_index, edge_attr, W0, a_src0, a_dst0, We0,
           W1, a_src1, a_dst1, We1):
    del size
    n = x.shape[0]
    src = edge_index[0]
    dst = edge_index[1]

    as0 = _attn_mat(a_src0)
    ad0 = _attn_mat(a_dst0)
    as1 = _attn_mat(a_src1)
    ad1 = _attn_mat(a_dst1)
    rows = jnp.arange(D)
    expm = ((rows[None, :] // DH) == jnp.arange(HP)[:, None]).astype(jnp.float32)
    zpad = jnp.zeros((We0.shape[0], HP - H), jnp.float32)
    we0p = jnp.concatenate([We0, zpad], axis=1)
    we1p = jnp.concatenate([We1, zpad], axis=1)
    zd = jnp.zeros((n, D), jnp.float32)
    zh = jnp.zeros((n, HP), jnp.float32)

    bn = 2000
    h0, als0, ald0 = _tc_pre(x, W0, as0, ad0, bn)
    e0, e1 = _tc_edge_proj(edge_attr, we0p, we1p, 8000)
    num0, den0 = _sc_edges(src, dst, h0, als0, ald0, e0, zd, zh)
    h1, als1, ald1 = _tc_mid(num0, den0, expm, W1, as1, ad1, bn, n)
    num1, den1 = _sc_edges(src, dst, h1, als1, ald1, e1, zd, zh)
    return _tc_post(num1, den1, expm, bn, n)


# speedup vs baseline: 75.7934x; 1.1042x over previous
import functools

import jax
import jax.numpy as jnp
from jax import lax
from jax.experimental import pallas as pl
from jax.experimental.pallas import tpu as pltpu
from jax.experimental.pallas import tpu_sc as plsc

H = 8
DH = 16
D = 128
HP = 16
NC = 2
NS = 16
NW = NC * NS
C = 80


def _pre_body(x_ref, w_ref, as_ref, ad_ref, h_ref, als_ref, ald_ref):
    h = jnp.dot(x_ref[...], w_ref[...], preferred_element_type=jnp.float32)
    h_ref[...] = h
    als_ref[...] = jnp.dot(h, as_ref[...], preferred_element_type=jnp.float32)
    ald_ref[...] = jnp.dot(h, ad_ref[...], preferred_element_type=jnp.float32)


def _tc_pre(x, w, a_s, a_d, bn):
    n = x.shape[0]
    grid = n // bn
    return pl.pallas_call(
        _pre_body,
        grid=(grid,),
        in_specs=[
            pl.BlockSpec((bn, D), lambda i: (i, 0)),
            pl.BlockSpec((D, D), lambda i: (0, 0)),
            pl.BlockSpec((D, HP), lambda i: (0, 0)),
            pl.BlockSpec((D, HP), lambda i: (0, 0)),
        ],
        out_specs=[
            pl.BlockSpec((bn, D), lambda i: (i, 0)),
            pl.BlockSpec((bn, HP), lambda i: (i, 0)),
            pl.BlockSpec((bn, HP), lambda i: (i, 0)),
        ],
        out_shape=[
            jax.ShapeDtypeStruct((n, D), jnp.float32),
            jax.ShapeDtypeStruct((n, HP), jnp.float32),
            jax.ShapeDtypeStruct((n, HP), jnp.float32),
        ],
    )(x, w, a_s, a_d)


def _e_body(ea_ref, w0_ref, w1_ref, e0_ref, e1_ref):
    ea = ea_ref[...]
    e0_ref[...] = jnp.dot(ea, w0_ref[...], preferred_element_type=jnp.float32)
    e1_ref[...] = jnp.dot(ea, w1_ref[...], preferred_element_type=jnp.float32)


def _tc_edge_proj(edge_attr, we0p, we1p, be):
    e_cnt, de = edge_attr.shape
    grid = e_cnt // be
    return pl.pallas_call(
        _e_body,
        grid=(grid,),
        in_specs=[
            pl.BlockSpec((be, de), lambda i: (i, 0)),
            pl.BlockSpec((de, HP), lambda i: (0, 0)),
            pl.BlockSpec((de, HP), lambda i: (0, 0)),
        ],
        out_specs=[
            pl.BlockSpec((be, HP), lambda i: (i, 0)),
            pl.BlockSpec((be, HP), lambda i: (i, 0)),
        ],
        out_shape=[
            jax.ShapeDtypeStruct((e_cnt, HP), jnp.float32),
            jax.ShapeDtypeStruct((e_cnt, HP), jnp.float32),
        ],
    )(edge_attr, we0p, we1p)


def _finish(num_ref, den_ref, exp_ref):
    numt = num_ref[0] + num_ref[1]
    dent = den_ref[0] + den_ref[1]
    dexp = jnp.dot(dent, exp_ref[...], preferred_element_type=jnp.float32)
    o = numt / (dexp + 1e-16)
    return jnp.where(o > 0, o, jnp.exp(o) - 1.0)


def _mid_body(num_ref, den_ref, exp_ref, w_ref, as_ref, ad_ref,
              h_ref, als_ref, ald_ref):
    x1 = _finish(num_ref, den_ref, exp_ref)
    h = jnp.dot(x1, w_ref[...], preferred_element_type=jnp.float32)
    h_ref[...] = h
    als_ref[...] = jnp.dot(h, as_ref[...], preferred_element_type=jnp.float32)
    ald_ref[...] = jnp.dot(h, ad_ref[...], preferred_element_type=jnp.float32)


def _tc_mid(num, den, expm, w, a_s, a_d, bn, n):
    grid = n // bn
    return pl.pallas_call(
        _mid_body,
        grid=(grid,),
        in_specs=[
            pl.BlockSpec((NC, bn, D), lambda i: (0, i, 0)),
            pl.BlockSpec((NC, bn, HP), lambda i: (0, i, 0)),
            pl.BlockSpec((HP, D), lambda i: (0, 0)),
            pl.BlockSpec((D, D), lambda i: (0, 0)),
            pl.BlockSpec((D, HP), lambda i: (0, 0)),
            pl.BlockSpec((D, HP), lambda i: (0, 0)),
        ],
        out_specs=[
            pl.BlockSpec((bn, D), lambda i: (i, 0)),
            pl.BlockSpec((bn, HP), lambda i: (i, 0)),
            pl.BlockSpec((bn, HP), lambda i: (i, 0)),
        ],
        out_shape=[
            jax.ShapeDtypeStruct((n, D), jnp.float32),
            jax.ShapeDtypeStruct((n, HP), jnp.float32),
            jax.ShapeDtypeStruct((n, HP), jnp.float32),
        ],
    )(num, den, expm, w, a_s, a_d)


def _post_body(num_ref, den_ref, exp_ref, out_ref):
    out_ref[...] = _finish(num_ref, den_ref, exp_ref)


def _tc_post(num, den, expm, bn, n):
    grid = n // bn
    return pl.pallas_call(
        _post_body,
        grid=(grid,),
        in_specs=[
            pl.BlockSpec((NC, bn, D), lambda i: (0, i, 0)),
            pl.BlockSpec((NC, bn, HP), lambda i: (0, i, 0)),
            pl.BlockSpec((HP, D), lambda i: (0, 0)),
        ],
        out_specs=pl.BlockSpec((bn, D), lambda i: (i, 0)),
        out_shape=jax.ShapeDtypeStruct((n, D), jnp.float32),
    )(num, den, expm)


def _sc_edges(src, dst, h, als, ald, e, zd, zh):
    n = zd.shape[0]
    e_cnt = src.shape[0]
    ew = e_cnt // NW
    nchunk = ew // C
    rt = (n // NS) // 8 * 8
    rem = n - NS * rt

    mesh = plsc.VectorSubcoreMesh(core_axis_name="c", subcore_axis_name="s")

    nring = 4
    scratch = [
        pltpu.VMEM((nring, C), jnp.int32),
        pltpu.VMEM((nring, C), jnp.int32),
        [pltpu.VMEM((C, HP), jnp.float32)] * 2,
        [pltpu.VMEM((C, HP), jnp.float32)] * 2,
        [pltpu.VMEM((C, HP), jnp.float32)] * 2,
        [pltpu.VMEM((C, HP), jnp.float32)] * 2,
        [pltpu.VMEM((C, D), jnp.float32)] * 2,
        pltpu.VMEM_SHARED((n, D), jnp.float32),
        pltpu.VMEM_SHARED((n, HP), jnp.float32),
        [pltpu.SemaphoreType.DMA] * 2,
        [pltpu.SemaphoreType.DMA] * 2,
        [pltpu.SemaphoreType.DMA] * nring,
    ]

    @functools.partial(
        pl.kernel,
        mesh=mesh,
        compiler_params=pltpu.CompilerParams(use_tc_tiling_on_sc=False),
        out_type=[
            jax.ShapeDtypeStruct((NC, n, D), jnp.float32),
            jax.ShapeDtypeStruct((NC, n, HP), jnp.float32),
        ],
        scratch_types=scratch,
    )
    def sc_kernel(src_hbm, dst_hbm, h_hbm, als_hbm, ald_hbm, e_hbm,
                  zd_hbm, zh_hbm, num_out, den_out,
                  src_v, dst_v, als_v, ald_v, e_v, ex_v, h_v,
                  num_sh, den_sh, gsem, ssem, isem):
        cid = lax.axis_index("c")
        sid = lax.axis_index("s")
        wid = sid * NC + cid
        base0 = wid * ew
        zrow = sid * rt

        def issue_idx(j, s):
            pltpu.async_copy(src_hbm.at[pl.ds(base0 + j * C, C)],
                             src_v.at[s], isem[s])
            pltpu.async_copy(dst_hbm.at[pl.ds(base0 + j * C, C)],
                             dst_v.at[s], isem[s])

        def wait_idx(s):
            pltpu.make_async_copy(src_hbm.at[pl.ds(base0, C)], src_v.at[s],
                                  isem[s]).wait()
            pltpu.make_async_copy(dst_hbm.at[pl.ds(base0, C)], dst_v.at[s],
                                  isem[s]).wait()

        issue_idx(0, 0)
        issue_idx(1, 1)
        issue_idx(2, 2)
        issue_idx(3, 3)
        pltpu.sync_copy(zd_hbm.at[pl.ds(zrow, rt)], num_sh.at[pl.ds(zrow, rt)])
        pltpu.sync_copy(zh_hbm.at[pl.ds(zrow, rt)], den_sh.at[pl.ds(zrow, rt)])

        @pl.when(sid == NS - 1)
        def _zero_tail():
            tr = NS * rt
            pltpu.sync_copy(zd_hbm.at[pl.ds(tr, rem)],
                            num_sh.at[pl.ds(tr, rem)])
            pltpu.sync_copy(zh_hbm.at[pl.ds(tr, rem)],
                            den_sh.at[pl.ds(tr, rem)])

        plsc.subcore_barrier()

        def issue_gathers(j, p, s):
            srow = src_v.at[s]
            pltpu.async_copy(als_hbm.at[srow], als_v[p], gsem[p])
            pltpu.async_copy(ald_hbm.at[dst_v.at[s]], ald_v[p], gsem[p])
            pltpu.async_copy(e_hbm.at[pl.ds(base0 + j * C, C)], e_v[p],
                             gsem[p])
            pltpu.async_copy(h_hbm.at[srow], h_v[p], gsem[p])

        def wait_gathers(p):
            pltpu.make_async_copy(als_hbm.at[src_v.at[0]], als_v[p],
                                  gsem[p]).wait()
            pltpu.make_async_copy(ald_hbm.at[dst_v.at[0]], ald_v[p],
                                  gsem[p]).wait()
            pltpu.make_async_copy(e_hbm.at[pl.ds(base0, C)], e_v[p],
                                  gsem[p]).wait()
            pltpu.make_async_copy(h_hbm.at[src_v.at[0]], h_v[p],
                                  gsem[p]).wait()

        kidx = [jnp.full((DH,), k, jnp.int32) for k in range(H)]

        def compute(p):
            def edge_body(c, carry2):
                lg = als_v[p][c] + ald_v[p][c] + e_v[p][c]
                lg = jnp.where(lg >= 0.0, lg, 0.2 * lg)
                exv = jnp.exp(lg)
                ex_v[p][c] = exv
                for k in range(H):
                    hk = h_v[p][c, pl.ds(k * DH, DH)]
                    mk = exv.at[kidx[k]].get(mode="promise_in_bounds")
                    h_v[p][c, pl.ds(k * DH, DH)] = hk * mk
                return carry2

            lax.fori_loop(0, C, edge_body, 0, unroll=2)

        def issue_scatters(p, s):
            drow = dst_v.at[s]
            pltpu.async_copy(ex_v[p], den_sh.at[drow], ssem[p], add=True)
            pltpu.async_copy(h_v[p], num_sh.at[drow], ssem[p], add=True)

        def wait_scatters(p):
            pltpu.make_async_copy(ex_v[p], den_sh.at[dst_v.at[0]],
                                  ssem[p]).wait()
            pltpu.make_async_copy(h_v[p], num_sh.at[dst_v.at[0]],
                                  ssem[p]).wait()

        wait_idx(0)
        issue_gathers(0, 0, 0)
        wait_idx(1)
        issue_gathers(1, 1, 1)
        wait_gathers(0)
        compute(0)
        issue_scatters(0, 0)

        def halfstep(j, pj, s_wait, s_issue, last_idx):
            wait_scatters((pj + 1) % 2)
            wait_idx(s_wait)
            issue_gathers(j + 1, (pj + 1) % 2, s_wait)
            if not last_idx:
                issue_idx(j + 3, s_issue)
            wait_gathers(pj)
            compute(pj)
            issue_scatters(pj, (s_wait - 1) % 4)

        halfstep(1, 1, 2, 0, False)

        def quad_body(u, carry):
            j = 4 * u + 2
            halfstep(j, 0, 3, 1, False)
            halfstep(j + 1, 1, 0, 2, False)
            halfstep(j + 2, 0, 1, 3, False)
            halfstep(j + 3, 1, 2, 0, False)
            return carry

        nquad = (nchunk - 5) // 4
        lax.fori_loop(0, nquad, quad_body, 0)

        halfstep(nchunk - 3, 0, 3, None, True)
        halfstep(nchunk - 2, 1, 0, None, True)
        wait_scatters(1)
        wait_gathers(0)
        compute(0)
        issue_scatters(0, 0)
        wait_scatters(0)

        plsc.subcore_barrier()
        pltpu.sync_copy(num_sh.at[pl.ds(zrow, rt)],
                        num_out.at[cid, pl.ds(zrow, rt)])
        pltpu.sync_copy(den_sh.at[pl.ds(zrow, rt)],
                        den_out.at[cid, pl.ds(zrow, rt)])

        @pl.when(sid == NS - 1)
        def _write_tail():
            tr = NS * rt
            pltpu.sync_copy(num_sh.at[pl.ds(tr, rem)],
                            num_out.at[cid, pl.ds(tr, rem)])
            pltpu.sync_copy(den_sh.at[pl.ds(tr, rem)],
                            den_out.at[cid, pl.ds(tr, rem)])

    return sc_kernel(src, dst, h, als, ald, e, zd, zh)


def _attn_mat(a):
    rows = jnp.arange(D)
    mask = (rows[:, None] // DH) == jnp.arange(HP)[None, :]
    return mask.astype(jnp.float32) * a.reshape(D)[:, None]


def kernel(x, size, edge_index, edge_attr, W0, a_src0, a_dst0, We0,
           W1, a_src1, a_dst1, We1):
    del size
    n = x.shape[0]
    src = edge_index[0]
    dst = edge_index[1]

    as0 = _attn_mat(a_src0)
    ad0 = _attn_mat(a_dst0)
    as1 = _attn_mat(a_src1)
    ad1 = _attn_mat(a_dst1)
    rows = jnp.arange(D)
    expm = ((rows[None, :] // DH) == jnp.arange(HP)[:, None]).astype(jnp.float32)
    zpad = jnp.zeros((We0.shape[0], HP - H), jnp.float32)
    we0p = jnp.concatenate([We0, zpad], axis=1)
    we1p = jnp.concatenate([We1, zpad], axis=1)
    zd = jnp.zeros((n, D), jnp.float32)
    zh = jnp.zeros((n, HP), jnp.float32)

    bn = 2000
    h0, als0, ald0 = _tc_pre(x, W0, as0, ad0, bn)
    e0, e1 = _tc_edge_proj(edge_attr, we0p, we1p, 8000)
    num0, den0 = _sc_edges(src, dst, h0, als0, ald0, e0, zd, zh)
    h1, als1, ald1 = _tc_mid(num0, den0, expm, W1, as1, ad1, bn, n)
    num1, den1 = _sc_edges(src, dst, h1, als1, ald1, e1, zd, zh)
    return _tc_post(num1, den1, expm, bn, n)


# speedup vs baseline: 86.8417x; 1.1458x over previous
import functools

import jax
import jax.numpy as jnp
from jax import lax
from jax.experimental import pallas as pl
from jax.experimental.pallas import tpu as pltpu
from jax.experimental.pallas import tpu_sc as plsc

H = 8
DH = 16
D = 128
HP = 16
NC = 2
NS = 16
NW = NC * NS
C = 80


def _pre_body(x_ref, w_ref, as_ref, ad_ref, h_ref, als_ref, ald_ref):
    h = jnp.dot(x_ref[...], w_ref[...], preferred_element_type=jnp.float32)
    h_ref[...] = h
    als_ref[...] = jnp.dot(h, as_ref[...], preferred_element_type=jnp.float32)
    ald_ref[...] = jnp.dot(h, ad_ref[...], preferred_element_type=jnp.float32)


def _tc_pre(x, w, a_s, a_d, bn):
    n = x.shape[0]
    grid = n // bn
    return pl.pallas_call(
        _pre_body,
        grid=(grid,),
        in_specs=[
            pl.BlockSpec((bn, D), lambda i: (i, 0)),
            pl.BlockSpec((D, D), lambda i: (0, 0)),
            pl.BlockSpec((D, HP), lambda i: (0, 0)),
            pl.BlockSpec((D, HP), lambda i: (0, 0)),
        ],
        out_specs=[
            pl.BlockSpec((bn, D), lambda i: (i, 0)),
            pl.BlockSpec((bn, HP), lambda i: (i, 0)),
            pl.BlockSpec((bn, HP), lambda i: (i, 0)),
        ],
        out_shape=[
            jax.ShapeDtypeStruct((n, D), jnp.float32),
            jax.ShapeDtypeStruct((n, HP), jnp.float32),
            jax.ShapeDtypeStruct((n, HP), jnp.float32),
        ],
    )(x, w, a_s, a_d)


def _e_body(ea_ref, w0_ref, w1_ref, e0_ref, e1_ref):
    ea = ea_ref[...]
    e0_ref[...] = jnp.dot(ea, w0_ref[...], preferred_element_type=jnp.float32)
    e1_ref[...] = jnp.dot(ea, w1_ref[...], preferred_element_type=jnp.float32)


def _tc_edge_proj(ea2, we0k, we1k, be):
    rows = ea2.shape[0]
    grid = rows // be
    return pl.pallas_call(
        _e_body,
        grid=(grid,),
        in_specs=[
            pl.BlockSpec((be, D), lambda i: (i, 0)),
            pl.BlockSpec((D, D), lambda i: (0, 0)),
            pl.BlockSpec((D, D), lambda i: (0, 0)),
        ],
        out_specs=[
            pl.BlockSpec((be, D), lambda i: (i, 0)),
            pl.BlockSpec((be, D), lambda i: (i, 0)),
        ],
        out_shape=[
            jax.ShapeDtypeStruct((rows, D), jnp.float32),
            jax.ShapeDtypeStruct((rows, D), jnp.float32),
        ],
    )(ea2, we0k, we1k)


def _finish(num_ref, den_ref, exp_ref):
    numt = num_ref[0] + num_ref[1]
    dent = den_ref[0] + den_ref[1]
    dexp = jnp.dot(dent, exp_ref[...], preferred_element_type=jnp.float32)
    o = numt / (dexp + 1e-16)
    return jnp.where(o > 0, o, jnp.exp(o) - 1.0)


def _mid_body(num_ref, den_ref, exp_ref, w_ref, as_ref, ad_ref,
              h_ref, als_ref, ald_ref):
    x1 = _finish(num_ref, den_ref, exp_ref)
    h = jnp.dot(x1, w_ref[...], preferred_element_type=jnp.float32)
    h_ref[...] = h
    als_ref[...] = jnp.dot(h, as_ref[...], preferred_element_type=jnp.float32)
    ald_ref[...] = jnp.dot(h, ad_ref[...], preferred_element_type=jnp.float32)


def _tc_mid(num, den, expm, w, a_s, a_d, bn, n):
    grid = n // bn
    return pl.pallas_call(
        _mid_body,
        grid=(grid,),
        in_specs=[
            pl.BlockSpec((NC, bn, D), lambda i: (0, i, 0)),
            pl.BlockSpec((NC, bn, HP), lambda i: (0, i, 0)),
            pl.BlockSpec((HP, D), lambda i: (0, 0)),
            pl.BlockSpec((D, D), lambda i: (0, 0)),
            pl.BlockSpec((D, HP), lambda i: (0, 0)),
            pl.BlockSpec((D, HP), lambda i: (0, 0)),
        ],
        out_specs=[
            pl.BlockSpec((bn, D), lambda i: (i, 0)),
            pl.BlockSpec((bn, HP), lambda i: (i, 0)),
            pl.BlockSpec((bn, HP), lambda i: (i, 0)),
        ],
        out_shape=[
            jax.ShapeDtypeStruct((n, D), jnp.float32),
            jax.ShapeDtypeStruct((n, HP), jnp.float32),
            jax.ShapeDtypeStruct((n, HP), jnp.float32),
        ],
    )(num, den, expm, w, a_s, a_d)


def _post_body(num_ref, den_ref, exp_ref, out_ref):
    out_ref[...] = _finish(num_ref, den_ref, exp_ref)


def _tc_post(num, den, expm, bn, n):
    grid = n // bn
    return pl.pallas_call(
        _post_body,
        grid=(grid,),
        in_specs=[
            pl.BlockSpec((NC, bn, D), lambda i: (0, i, 0)),
            pl.BlockSpec((NC, bn, HP), lambda i: (0, i, 0)),
            pl.BlockSpec((HP, D), lambda i: (0, 0)),
        ],
        out_specs=pl.BlockSpec((bn, D), lambda i: (i, 0)),
        out_shape=jax.ShapeDtypeStruct((n, D), jnp.float32),
    )(num, den, expm)


def _sc_edges(src, dst, h, als, ald, e, zd, zh):
    n = zd.shape[0]
    e_cnt = src.shape[0]
    ew = e_cnt // NW
    nchunk = ew // C
    rt = (n // NS) // 8 * 8
    rem = n - NS * rt

    mesh = plsc.VectorSubcoreMesh(core_axis_name="c", subcore_axis_name="s")

    nring = 4
    scratch = [
        pltpu.VMEM((nring, C), jnp.int32),
        pltpu.VMEM((nring, C), jnp.int32),
        [pltpu.VMEM((C, HP), jnp.float32)] * 2,
        [pltpu.VMEM((C, HP), jnp.float32)] * 2,
        [pltpu.VMEM((C // 8, D), jnp.float32)] * 2,
        [pltpu.VMEM((C, HP), jnp.float32)] * 2,
        [pltpu.VMEM((C, D), jnp.float32)] * 2,
        pltpu.VMEM_SHARED((n, D), jnp.float32),
        pltpu.VMEM_SHARED((n, HP), jnp.float32),
        [pltpu.SemaphoreType.DMA] * 2,
        [pltpu.SemaphoreType.DMA] * 2,
        [pltpu.SemaphoreType.DMA] * nring,
    ]

    @functools.partial(
        pl.kernel,
        mesh=mesh,
        compiler_params=pltpu.CompilerParams(use_tc_tiling_on_sc=False),
        out_type=[
            jax.ShapeDtypeStruct((NC, n, D), jnp.float32),
            jax.ShapeDtypeStruct((NC, n, HP), jnp.float32),
        ],
        scratch_types=scratch,
    )
    def sc_kernel(src_hbm, dst_hbm, h_hbm, als_hbm, ald_hbm, e_hbm,
                  zd_hbm, zh_hbm, num_out, den_out,
                  src_v, dst_v, als_v, ald_v, e_v, ex_v, h_v,
                  num_sh, den_sh, gsem, ssem, isem):
        cid = lax.axis_index("c")
        sid = lax.axis_index("s")
        wid = sid * NC + cid
        base0 = wid * ew
        zrow = sid * rt

        def issue_idx(j, s):
            pltpu.async_copy(src_hbm.at[pl.ds(base0 + j * C, C)],
                             src_v.at[s], isem[s])
            pltpu.async_copy(dst_hbm.at[pl.ds(base0 + j * C, C)],
                             dst_v.at[s], isem[s])

        def wait_idx(s):
            pltpu.make_async_copy(src_hbm.at[pl.ds(base0, C)], src_v.at[s],
                                  isem[s]).wait()
            pltpu.make_async_copy(dst_hbm.at[pl.ds(base0, C)], dst_v.at[s],
                                  isem[s]).wait()

        issue_idx(0, 0)
        issue_idx(1, 1)
        issue_idx(2, 2)
        issue_idx(3, 3)
        pltpu.sync_copy(zd_hbm.at[pl.ds(zrow, rt)], num_sh.at[pl.ds(zrow, rt)])
        pltpu.sync_copy(zh_hbm.at[pl.ds(zrow, rt)], den_sh.at[pl.ds(zrow, rt)])

        @pl.when(sid == NS - 1)
        def _zero_tail():
            tr = NS * rt
            pltpu.sync_copy(zd_hbm.at[pl.ds(tr, rem)],
                            num_sh.at[pl.ds(tr, rem)])
            pltpu.sync_copy(zh_hbm.at[pl.ds(tr, rem)],
                            den_sh.at[pl.ds(tr, rem)])

        plsc.subcore_barrier()

        ec = C // 8
        eb0 = base0 // 8

        def issue_gathers(j, p, s):
            srow = src_v.at[s]
            pltpu.async_copy(als_hbm.at[srow], als_v[p], gsem[p])
            pltpu.async_copy(ald_hbm.at[dst_v.at[s]], ald_v[p], gsem[p])
            pltpu.async_copy(e_hbm.at[pl.ds(eb0 + j * ec, ec)], e_v[p],
                             gsem[p])
            pltpu.async_copy(h_hbm.at[srow], h_v[p], gsem[p])

        def wait_gathers(p):
            pltpu.make_async_copy(als_hbm.at[src_v.at[0]], als_v[p],
                                  gsem[p]).wait()
            pltpu.make_async_copy(ald_hbm.at[dst_v.at[0]], ald_v[p],
                                  gsem[p]).wait()
            pltpu.make_async_copy(e_hbm.at[pl.ds(eb0, ec)], e_v[p],
                                  gsem[p]).wait()
            pltpu.make_async_copy(h_hbm.at[src_v.at[0]], h_v[p],
                                  gsem[p]).wait()

        kidx = [jnp.full((DH,), k, jnp.int32) for k in range(H)]

        def compute(p):
            def edge_body(r, carry2):
                for jj in range(8):
                    c = r * 8 + jj
                    lg = (als_v[p][c] + ald_v[p][c]
                          + e_v[p][r, pl.ds(jj * DH, DH)])
                    lg = jnp.where(lg >= 0.0, lg, 0.2 * lg)
                    exv = jnp.exp(lg)
                    ex_v[p][c] = exv
                    for k in range(H):
                        hk = h_v[p][c, pl.ds(k * DH, DH)]
                        mk = exv.at[kidx[k]].get(mode="promise_in_bounds")
                        h_v[p][c, pl.ds(k * DH, DH)] = hk * mk
                return carry2

            lax.fori_loop(0, C // 8, edge_body, 0)

        def issue_scatters(p, s):
            drow = dst_v.at[s]
            pltpu.async_copy(ex_v[p], den_sh.at[drow], ssem[p], add=True)
            pltpu.async_copy(h_v[p], num_sh.at[drow], ssem[p], add=True)

        def wait_scatters(p):
            pltpu.make_async_copy(ex_v[p], den_sh.at[dst_v.at[0]],
                                  ssem[p]).wait()
            pltpu.make_async_copy(h_v[p], num_sh.at[dst_v.at[0]],
                                  ssem[p]).wait()

        wait_idx(0)
        issue_gathers(0, 0, 0)
        wait_idx(1)
        issue_gathers(1, 1, 1)
        wait_gathers(0)
        compute(0)
        issue_scatters(0, 0)

        def halfstep(j, pj, s_wait, s_issue, last_idx):
            wait_scatters((pj + 1) % 2)
            wait_idx(s_wait)
            issue_gathers(j + 1, (pj + 1) % 2, s_wait)
            if not last_idx:
                issue_idx(j + 3, s_issue)
            wait_gathers(pj)
            compute(pj)
            issue_scatters(pj, (s_wait - 1) % 4)

        halfstep(1, 1, 2, 0, False)

        def quad_body(u, carry):
            j = 4 * u + 2
            halfstep(j, 0, 3, 1, False)
            halfstep(j + 1, 1, 0, 2, False)
            halfstep(j + 2, 0, 1, 3, False)
            halfstep(j + 3, 1, 2, 0, False)
            return carry

        nquad = (nchunk - 5) // 4
        lax.fori_loop(0, nquad, quad_body, 0)

        halfstep(nchunk - 3, 0, 3, None, True)
        halfstep(nchunk - 2, 1, 0, None, True)
        wait_scatters(1)
        wait_gathers(0)
        compute(0)
        issue_scatters(0, 0)
        wait_scatters(0)

        plsc.subcore_barrier()
        pltpu.sync_copy(num_sh.at[pl.ds(zrow, rt)],
                        num_out.at[cid, pl.ds(zrow, rt)])
        pltpu.sync_copy(den_sh.at[pl.ds(zrow, rt)],
                        den_out.at[cid, pl.ds(zrow, rt)])

        @pl.when(sid == NS - 1)
        def _write_tail():
            tr = NS * rt
            pltpu.sync_copy(num_sh.at[pl.ds(tr, rem)],
                            num_out.at[cid, pl.ds(tr, rem)])
            pltpu.sync_copy(den_sh.at[pl.ds(tr, rem)],
                            den_out.at[cid, pl.ds(tr, rem)])

    return sc_kernel(src, dst, h, als, ald, e, zd, zh)


def _attn_mat(a):
    rows = jnp.arange(D)
    mask = (rows[:, None] // DH) == jnp.arange(HP)[None, :]
    return mask.astype(jnp.float32) * a.reshape(D)[:, None]


def kernel(x, size, edge_index, edge_attr, W0, a_src0, a_dst0, We0,
           W1, a_src1, a_dst1, We1):
    del size
    n = x.shape[0]
    src = edge_index[0]
    dst = edge_index[1]

    as0 = _attn_mat(a_src0)
    ad0 = _attn_mat(a_dst0)
    as1 = _attn_mat(a_src1)
    ad1 = _attn_mat(a_dst1)
    rows = jnp.arange(D)
    expm = ((rows[None, :] // DH) == jnp.arange(HP)[:, None]).astype(jnp.float32)
    zpad = jnp.zeros((We0.shape[0], HP - H), jnp.float32)
    we0p = jnp.concatenate([We0, zpad], axis=1)
    we1p = jnp.concatenate([We1, zpad], axis=1)
    eye8 = jnp.eye(8, dtype=jnp.float32)
    we0k = jnp.kron(eye8, we0p)
    we1k = jnp.kron(eye8, we1p)
    ea2 = edge_attr.reshape(-1, 8 * edge_attr.shape[1])
    zd = jnp.zeros((n, D), jnp.float32)
    zh = jnp.zeros((n, HP), jnp.float32)

    bn = 2000
    h0, als0, ald0 = _tc_pre(x, W0, as0, ad0, bn)
    e0, e1 = _tc_edge_proj(ea2, we0k, we1k, 1000)
    num0, den0 = _sc_edges(src, dst, h0, als0, ald0, e0, zd, zh)
    h1, als1, ald1 = _tc_mid(num0, den0, expm, W1, as1, ad1, bn, n)
    num1, den1 = _sc_edges(src, dst, h1, als1, ald1, e1, zd, zh)
    return _tc_post(num1, den1, expm, bn, n)


# speedup vs baseline: 87.4167x; 1.0066x over previous
import functools

import jax
import jax.numpy as jnp
from jax import lax
from jax.experimental import pallas as pl
from jax.experimental.pallas import tpu as pltpu
from jax.experimental.pallas import tpu_sc as plsc

H = 8
DH = 16
D = 128
HP = 16
NC = 2
NS = 16
NW = NC * NS
C = 80


def _pre_body(x_ref, w_ref, as_ref, ad_ref, h_ref, als_ref, ald_ref):
    h = jnp.dot(x_ref[...], w_ref[...], preferred_element_type=jnp.float32)
    h_ref[...] = h
    als_ref[...] = jnp.dot(h, as_ref[...], preferred_element_type=jnp.float32)
    ald_ref[...] = jnp.dot(h, ad_ref[...], preferred_element_type=jnp.float32)


def _tc_pre(x, w, a_s, a_d, bn):
    n = x.shape[0]
    grid = n // bn
    return pl.pallas_call(
        _pre_body,
        grid=(grid,),
        in_specs=[
            pl.BlockSpec((bn, D), lambda i: (i, 0)),
            pl.BlockSpec((D, D), lambda i: (0, 0)),
            pl.BlockSpec((D, HP), lambda i: (0, 0)),
            pl.BlockSpec((D, HP), lambda i: (0, 0)),
        ],
        out_specs=[
            pl.BlockSpec((bn, D), lambda i: (i, 0)),
            pl.BlockSpec((bn, HP), lambda i: (i, 0)),
            pl.BlockSpec((bn, HP), lambda i: (i, 0)),
        ],
        out_shape=[
            jax.ShapeDtypeStruct((n, D), jnp.float32),
            jax.ShapeDtypeStruct((n, HP), jnp.float32),
            jax.ShapeDtypeStruct((n, HP), jnp.float32),
        ],
    )(x, w, a_s, a_d)


def _e_body(ea_ref, w_ref, e_ref):
    e_ref[...] = jnp.dot(ea_ref[...], w_ref[...],
                         preferred_element_type=jnp.float32)


def _tc_edge_proj(ea2, wek, be):
    rows = ea2.shape[0]
    grid = rows // be
    return pl.pallas_call(
        _e_body,
        grid=(grid,),
        in_specs=[
            pl.BlockSpec((be, D), lambda i: (i, 0)),
            pl.BlockSpec((D, D), lambda i: (0, 0)),
        ],
        out_specs=pl.BlockSpec((be, D), lambda i: (i, 0)),
        out_shape=jax.ShapeDtypeStruct((rows, D), jnp.float32),
    )(ea2, wek)


def _finish(num_ref, den_ref, exp_ref):
    numt = num_ref[0] + num_ref[1]
    dent = den_ref[0] + den_ref[1]
    dexp = jnp.dot(dent, exp_ref[...], preferred_element_type=jnp.float32)
    o = numt / (dexp + 1e-16)
    return jnp.where(o > 0, o, jnp.exp(o) - 1.0)


def _mid_body(num_ref, den_ref, exp_ref, w_ref, as_ref, ad_ref,
              h_ref, als_ref, ald_ref):
    x1 = _finish(num_ref, den_ref, exp_ref)
    h = jnp.dot(x1, w_ref[...], preferred_element_type=jnp.float32)
    h_ref[...] = h
    als_ref[...] = jnp.dot(h, as_ref[...], preferred_element_type=jnp.float32)
    ald_ref[...] = jnp.dot(h, ad_ref[...], preferred_element_type=jnp.float32)


def _tc_mid(num, den, expm, w, a_s, a_d, bn, n):
    grid = n // bn
    return pl.pallas_call(
        _mid_body,
        grid=(grid,),
        in_specs=[
            pl.BlockSpec((NC, bn, D), lambda i: (0, i, 0)),
            pl.BlockSpec((NC, bn, HP), lambda i: (0, i, 0)),
            pl.BlockSpec((HP, D), lambda i: (0, 0)),
            pl.BlockSpec((D, D), lambda i: (0, 0)),
            pl.BlockSpec((D, HP), lambda i: (0, 0)),
            pl.BlockSpec((D, HP), lambda i: (0, 0)),
        ],
        out_specs=[
            pl.BlockSpec((bn, D), lambda i: (i, 0)),
            pl.BlockSpec((bn, HP), lambda i: (i, 0)),
            pl.BlockSpec((bn, HP), lambda i: (i, 0)),
        ],
        out_shape=[
            jax.ShapeDtypeStruct((n, D), jnp.float32),
            jax.ShapeDtypeStruct((n, HP), jnp.float32),
            jax.ShapeDtypeStruct((n, HP), jnp.float32),
        ],
    )(num, den, expm, w, a_s, a_d)


def _post_body(num_ref, den_ref, exp_ref, out_ref):
    out_ref[...] = _finish(num_ref, den_ref, exp_ref)


def _tc_post(num, den, expm, bn, n):
    grid = n // bn
    return pl.pallas_call(
        _post_body,
        grid=(grid,),
        in_specs=[
            pl.BlockSpec((NC, bn, D), lambda i: (0, i, 0)),
            pl.BlockSpec((NC, bn, HP), lambda i: (0, i, 0)),
            pl.BlockSpec((HP, D), lambda i: (0, 0)),
        ],
        out_specs=pl.BlockSpec((bn, D), lambda i: (i, 0)),
        out_shape=jax.ShapeDtypeStruct((n, D), jnp.float32),
    )(num, den, expm)


def _sc_edges(src, dst, h, als, ald, e, zd, zh):
    n = zd.shape[0]
    e_cnt = src.shape[0]
    ew = e_cnt // NW
    nchunk = ew // C
    rt = (n // NS) // 8 * 8
    rem = n - NS * rt

    mesh = plsc.VectorSubcoreMesh(core_axis_name="c", subcore_axis_name="s")

    nring = 4
    scratch = [
        pltpu.VMEM((nring, C), jnp.int32),
        pltpu.VMEM((nring, C), jnp.int32),
        [pltpu.VMEM((C, HP), jnp.float32)] * 2,
        [pltpu.VMEM((C, HP), jnp.float32)] * 2,
        [pltpu.VMEM((C // 8, D), jnp.float32)] * 2,
        [pltpu.VMEM((C, HP), jnp.float32)] * 2,
        [pltpu.VMEM((C, D), jnp.float32)] * 2,
        pltpu.VMEM_SHARED((n, D), jnp.float32),
        pltpu.VMEM_SHARED((n, HP), jnp.float32),
        [pltpu.SemaphoreType.DMA] * 2,
        [pltpu.SemaphoreType.DMA] * 2,
        [pltpu.SemaphoreType.DMA] * nring,
    ]

    @functools.partial(
        pl.kernel,
        mesh=mesh,
        compiler_params=pltpu.CompilerParams(use_tc_tiling_on_sc=False),
        out_type=[
            jax.ShapeDtypeStruct((NC, n, D), jnp.float32),
            jax.ShapeDtypeStruct((NC, n, HP), jnp.float32),
        ],
        scratch_types=scratch,
    )
    def sc_kernel(src_hbm, dst_hbm, h_hbm, als_hbm, ald_hbm, e_hbm,
                  zd_hbm, zh_hbm, num_out, den_out,
                  src_v, dst_v, als_v, ald_v, e_v, ex_v, h_v,
                  num_sh, den_sh, gsem, ssem, isem):
        cid = lax.axis_index("c")
        sid = lax.axis_index("s")
        wid = sid * NC + cid
        base0 = wid * ew
        zrow = sid * rt

        def issue_idx(j, s):
            pltpu.async_copy(src_hbm.at[pl.ds(base0 + j * C, C)],
                             src_v.at[s], isem[s])
            pltpu.async_copy(dst_hbm.at[pl.ds(base0 + j * C, C)],
                             dst_v.at[s], isem[s])

        def wait_idx(s):
            pltpu.make_async_copy(src_hbm.at[pl.ds(base0, C)], src_v.at[s],
                                  isem[s]).wait()
            pltpu.make_async_copy(dst_hbm.at[pl.ds(base0, C)], dst_v.at[s],
                                  isem[s]).wait()

        issue_idx(0, 0)
        issue_idx(1, 1)
        issue_idx(2, 2)
        issue_idx(3, 3)
        pltpu.sync_copy(zd_hbm.at[pl.ds(zrow, rt)], num_sh.at[pl.ds(zrow, rt)])
        pltpu.sync_copy(zh_hbm.at[pl.ds(zrow, rt)], den_sh.at[pl.ds(zrow, rt)])

        @pl.when(sid == NS - 1)
        def _zero_tail():
            tr = NS * rt
            pltpu.sync_copy(zd_hbm.at[pl.ds(tr, rem)],
                            num_sh.at[pl.ds(tr, rem)])
            pltpu.sync_copy(zh_hbm.at[pl.ds(tr, rem)],
                            den_sh.at[pl.ds(tr, rem)])

        plsc.subcore_barrier()

        ec = C // 8
        eb0 = base0 // 8

        def issue_gathers(j, p, s):
            srow = src_v.at[s]
            pltpu.async_copy(als_hbm.at[srow], als_v[p], gsem[p])
            pltpu.async_copy(ald_hbm.at[dst_v.at[s]], ald_v[p], gsem[p])
            pltpu.async_copy(e_hbm.at[pl.ds(eb0 + j * ec, ec)], e_v[p],
                             gsem[p])
            pltpu.async_copy(h_hbm.at[srow], h_v[p], gsem[p])

        def wait_gathers(p):
            pltpu.make_async_copy(als_hbm.at[src_v.at[0]], als_v[p],
                                  gsem[p]).wait()
            pltpu.make_async_copy(ald_hbm.at[dst_v.at[0]], ald_v[p],
                                  gsem[p]).wait()
            pltpu.make_async_copy(e_hbm.at[pl.ds(eb0, ec)], e_v[p],
                                  gsem[p]).wait()
            pltpu.make_async_copy(h_hbm.at[src_v.at[0]], h_v[p],
                                  gsem[p]).wait()

        kidx = [jnp.full((DH,), k, jnp.int32) for k in range(H)]

        def compute(p):
            def edge_body(r, carry2):
                for jj in range(8):
                    c = r * 8 + jj
                    lg = (als_v[p][c] + ald_v[p][c]
                          + e_v[p][r, pl.ds(jj * DH, DH)])
                    lg = jnp.where(lg >= 0.0, lg, 0.2 * lg)
                    exv = jnp.exp(lg)
                    ex_v[p][c] = exv
                    for k in range(H):
                        hk = h_v[p][c, pl.ds(k * DH, DH)]
                        mk = exv.at[kidx[k]].get(mode="promise_in_bounds")
                        h_v[p][c, pl.ds(k * DH, DH)] = hk * mk
                return carry2

            lax.fori_loop(0, C // 8, edge_body, 0)

        def issue_scatters(p, s):
            drow = dst_v.at[s]
            pltpu.async_copy(ex_v[p], den_sh.at[drow], ssem[p], add=True)
            pltpu.async_copy(h_v[p], num_sh.at[drow], ssem[p], add=True)

        def wait_scatters(p):
            pltpu.make_async_copy(ex_v[p], den_sh.at[dst_v.at[0]],
                                  ssem[p]).wait()
            pltpu.make_async_copy(h_v[p], num_sh.at[dst_v.at[0]],
                                  ssem[p]).wait()

        wait_idx(0)
        issue_gathers(0, 0, 0)
        wait_idx(1)
        issue_gathers(1, 1, 1)
        wait_gathers(0)
        compute(0)
        issue_scatters(0, 0)

        def halfstep(j, pj, s_wait, s_issue, last_idx):
            wait_scatters((pj + 1) % 2)
            wait_idx(s_wait)
            issue_gathers(j + 1, (pj + 1) % 2, s_wait)
            if not last_idx:
                issue_idx(j + 3, s_issue)
            wait_gathers(pj)
            compute(pj)
            issue_scatters(pj, (s_wait - 1) % 4)

        halfstep(1, 1, 2, 0, False)

        def quad_body(u, carry):
            j = 4 * u + 2
            halfstep(j, 0, 3, 1, False)
            halfstep(j + 1, 1, 0, 2, False)
            halfstep(j + 2, 0, 1, 3, False)
            halfstep(j + 3, 1, 2, 0, False)
            return carry

        nquad = (nchunk - 5) // 4
        lax.fori_loop(0, nquad, quad_body, 0)

        halfstep(nchunk - 3, 0, 3, None, True)
        halfstep(nchunk - 2, 1, 0, None, True)
        wait_scatters(1)
        wait_gathers(0)
        compute(0)
        issue_scatters(0, 0)
        wait_scatters(0)

        plsc.subcore_barrier()
        pltpu.sync_copy(num_sh.at[pl.ds(zrow, rt)],
                        num_out.at[cid, pl.ds(zrow, rt)])
        pltpu.sync_copy(den_sh.at[pl.ds(zrow, rt)],
                        den_out.at[cid, pl.ds(zrow, rt)])

        @pl.when(sid == NS - 1)
        def _write_tail():
            tr = NS * rt
            pltpu.sync_copy(num_sh.at[pl.ds(tr, rem)],
                            num_out.at[cid, pl.ds(tr, rem)])
            pltpu.sync_copy(den_sh.at[pl.ds(tr, rem)],
                            den_out.at[cid, pl.ds(tr, rem)])

    return sc_kernel(src, dst, h, als, ald, e, zd, zh)


def _attn_mat(a):
    rows = jnp.arange(D)
    mask = (rows[:, None] // DH) == jnp.arange(HP)[None, :]
    return mask.astype(jnp.float32) * a.reshape(D)[:, None]


def kernel(x, size, edge_index, edge_attr, W0, a_src0, a_dst0, We0,
           W1, a_src1, a_dst1, We1):
    del size
    n = x.shape[0]
    src = edge_index[0]
    dst = edge_index[1]

    as0 = _attn_mat(a_src0)
    ad0 = _attn_mat(a_dst0)
    as1 = _attn_mat(a_src1)
    ad1 = _attn_mat(a_dst1)
    rows = jnp.arange(D)
    expm = ((rows[None, :] // DH) == jnp.arange(HP)[:, None]).astype(jnp.float32)
    zpad = jnp.zeros((We0.shape[0], HP - H), jnp.float32)
    we0p = jnp.concatenate([We0, zpad], axis=1)
    we1p = jnp.concatenate([We1, zpad], axis=1)
    eye8 = jnp.eye(8, dtype=jnp.float32)
    we0k = jnp.kron(eye8, we0p)
    we1k = jnp.kron(eye8, we1p)
    ea2 = edge_attr.reshape(-1, 8 * edge_attr.shape[1])
    zd = jnp.zeros((n, D), jnp.float32)
    zh = jnp.zeros((n, HP), jnp.float32)

    bn = 2000
    h0, als0, ald0 = _tc_pre(x, W0, as0, ad0, bn)
    e0 = _tc_edge_proj(ea2, we0k, 1000)
    e1 = _tc_edge_proj(ea2, we1k, 1000)
    num0, den0 = _sc_edges(src, dst, h0, als0, ald0, e0, zd, zh)
    h1, als1, ald1 = _tc_mid(num0, den0, expm, W1, as1, ad1, bn, n)
    num1, den1 = _sc_edges(src, dst, h1, als1, ald1, e1, zd, zh)
    return _tc_post(num1, den1, expm, bn, n)


# speedup vs baseline: 91.3010x; 1.0444x over previous
import functools

import jax
import jax.numpy as jnp
from jax import lax
from jax.experimental import pallas as pl
from jax.experimental.pallas import tpu as pltpu
from jax.experimental.pallas import tpu_sc as plsc

H = 8
DH = 16
D = 128
HP = 16
NC = 2
NS = 16
NW = NC * NS
C = 80


def _pre_body(x_ref, w_ref, as_ref, ad_ref, h_ref, als_ref, ald_ref):
    h = jnp.dot(x_ref[...], w_ref[...], preferred_element_type=jnp.float32)
    h_ref[...] = h
    als_ref[...] = jnp.dot(h, as_ref[...], preferred_element_type=jnp.float32)
    ald_ref[...] = jnp.dot(h, ad_ref[...], preferred_element_type=jnp.float32)


def _tc_pre(x, w, a_s, a_d, bn):
    n = x.shape[0]
    grid = n // bn
    return pl.pallas_call(
        _pre_body,
        grid=(grid,),
        in_specs=[
            pl.BlockSpec((bn, D), lambda i: (i, 0)),
            pl.BlockSpec((D, D), lambda i: (0, 0)),
            pl.BlockSpec((D, HP), lambda i: (0, 0)),
            pl.BlockSpec((D, HP), lambda i: (0, 0)),
        ],
        out_specs=[
            pl.BlockSpec((bn, D), lambda i: (i, 0)),
            pl.BlockSpec((bn, HP), lambda i: (i, 0)),
            pl.BlockSpec((bn, HP), lambda i: (i, 0)),
        ],
        out_shape=[
            jax.ShapeDtypeStruct((n, D), jnp.float32),
            jax.ShapeDtypeStruct((n, HP), jnp.float32),
            jax.ShapeDtypeStruct((n, HP), jnp.float32),
        ],
    )(x, w, a_s, a_d)


def _e_body(ea_ref, w_ref, e_ref):
    e_ref[...] = jnp.dot(ea_ref[...], w_ref[...],
                         preferred_element_type=jnp.float32)


def _tc_edge_proj(ea2, wek, be):
    rows = ea2.shape[0]
    grid = rows // be
    return pl.pallas_call(
        _e_body,
        grid=(grid,),
        in_specs=[
            pl.BlockSpec((be, D), lambda i: (i, 0)),
            pl.BlockSpec((D, D), lambda i: (0, 0)),
        ],
        out_specs=pl.BlockSpec((be, D), lambda i: (i, 0)),
        out_shape=jax.ShapeDtypeStruct((rows, D), jnp.float32),
    )(ea2, wek)


def _finish(num_ref, den_ref, exp_ref):
    numt = num_ref[0] + num_ref[1]
    dent = den_ref[0] + den_ref[1]
    dexp = jnp.dot(dent, exp_ref[...], preferred_element_type=jnp.float32)
    o = numt / (dexp + 1e-16)
    return jnp.where(o > 0, o, jnp.exp(o) - 1.0)


def _mid_body(num_ref, den_ref, exp_ref, w_ref, as_ref, ad_ref,
              h_ref, als_ref, ald_ref):
    x1 = _finish(num_ref, den_ref, exp_ref)
    h = jnp.dot(x1, w_ref[...], preferred_element_type=jnp.float32)
    h_ref[...] = h
    als_ref[...] = jnp.dot(h, as_ref[...], preferred_element_type=jnp.float32)
    ald_ref[...] = jnp.dot(h, ad_ref[...], preferred_element_type=jnp.float32)


def _tc_mid(num, den, expm, w, a_s, a_d, bn, n):
    grid = n // bn
    return pl.pallas_call(
        _mid_body,
        grid=(grid,),
        in_specs=[
            pl.BlockSpec((NC, bn, D), lambda i: (0, i, 0)),
            pl.BlockSpec((NC, bn, HP), lambda i: (0, i, 0)),
            pl.BlockSpec((HP, D), lambda i: (0, 0)),
            pl.BlockSpec((D, D), lambda i: (0, 0)),
            pl.BlockSpec((D, HP), lambda i: (0, 0)),
            pl.BlockSpec((D, HP), lambda i: (0, 0)),
        ],
        out_specs=[
            pl.BlockSpec((bn, D), lambda i: (i, 0)),
            pl.BlockSpec((bn, HP), lambda i: (i, 0)),
            pl.BlockSpec((bn, HP), lambda i: (i, 0)),
        ],
        out_shape=[
            jax.ShapeDtypeStruct((n, D), jnp.float32),
            jax.ShapeDtypeStruct((n, HP), jnp.float32),
            jax.ShapeDtypeStruct((n, HP), jnp.float32),
        ],
    )(num, den, expm, w, a_s, a_d)


def _post_body(num_ref, den_ref, exp_ref, out_ref):
    out_ref[...] = _finish(num_ref, den_ref, exp_ref)


def _tc_post(num, den, expm, bn, n):
    grid = n // bn
    return pl.pallas_call(
        _post_body,
        grid=(grid,),
        in_specs=[
            pl.BlockSpec((NC, bn, D), lambda i: (0, i, 0)),
            pl.BlockSpec((NC, bn, HP), lambda i: (0, i, 0)),
            pl.BlockSpec((HP, D), lambda i: (0, 0)),
        ],
        out_specs=pl.BlockSpec((bn, D), lambda i: (i, 0)),
        out_shape=jax.ShapeDtypeStruct((n, D), jnp.float32),
    )(num, den, expm)


def _sc_edges(src, dst, h, als, ald, e, zd, zh):
    n = zd.shape[0]
    e_cnt = src.shape[0]
    ew = e_cnt // NW
    nchunk = ew // C
    rt = (n // NS) // 8 * 8
    rem = n - NS * rt

    mesh = plsc.VectorSubcoreMesh(core_axis_name="c", subcore_axis_name="s")

    nring = 4
    scratch = [
        pltpu.VMEM((nring, C), jnp.int32),
        pltpu.VMEM((nring, C), jnp.int32),
        [pltpu.VMEM((C, HP), jnp.float32)] * 2,
        [pltpu.VMEM((C, HP), jnp.float32)] * 2,
        [pltpu.VMEM((C // 8, D), jnp.float32)] * 2,
        [pltpu.VMEM((C, HP), jnp.float32)] * 2,
        [pltpu.VMEM((C, D), jnp.float32)] * 2,
        pltpu.VMEM_SHARED((n, D), jnp.float32),
        pltpu.VMEM_SHARED((n, HP), jnp.float32),
        [pltpu.SemaphoreType.DMA] * 2,
        [pltpu.SemaphoreType.DMA] * 2,
        [pltpu.SemaphoreType.DMA] * nring,
    ]

    @functools.partial(
        pl.kernel,
        mesh=mesh,
        compiler_params=pltpu.CompilerParams(use_tc_tiling_on_sc=False),
        out_type=[
            jax.ShapeDtypeStruct((NC, n, D), jnp.float32),
            jax.ShapeDtypeStruct((NC, n, HP), jnp.float32),
        ],
        scratch_types=scratch,
    )
    def sc_kernel(src_hbm, dst_hbm, h_hbm, als_hbm, ald_hbm, e_hbm,
                  zd_hbm, zh_hbm, num_out, den_out,
                  src_v, dst_v, als_v, ald_v, e_v, ex_v, h_v,
                  num_sh, den_sh, gsem, ssem, isem):
        cid = lax.axis_index("c")
        sid = lax.axis_index("s")
        wid = sid * NC + cid
        base0 = wid * ew
        zrow = sid * rt

        def issue_idx(j, s):
            pltpu.async_copy(src_hbm.at[pl.ds(base0 + j * C, C)],
                             src_v.at[s], isem[s])
            pltpu.async_copy(dst_hbm.at[pl.ds(base0 + j * C, C)],
                             dst_v.at[s], isem[s])

        def wait_idx(s):
            pltpu.make_async_copy(src_hbm.at[pl.ds(base0, C)], src_v.at[s],
                                  isem[s]).wait()
            pltpu.make_async_copy(dst_hbm.at[pl.ds(base0, C)], dst_v.at[s],
                                  isem[s]).wait()

        issue_idx(0, 0)
        issue_idx(1, 1)
        issue_idx(2, 2)
        issue_idx(3, 3)
        pltpu.sync_copy(zd_hbm.at[pl.ds(zrow, rt)], num_sh.at[pl.ds(zrow, rt)])
        pltpu.sync_copy(zh_hbm.at[pl.ds(zrow, rt)], den_sh.at[pl.ds(zrow, rt)])

        @pl.when(sid == NS - 1)
        def _zero_tail():
            tr = NS * rt
            pltpu.sync_copy(zd_hbm.at[pl.ds(tr, rem)],
                            num_sh.at[pl.ds(tr, rem)])
            pltpu.sync_copy(zh_hbm.at[pl.ds(tr, rem)],
                            den_sh.at[pl.ds(tr, rem)])

        plsc.subcore_barrier()

        ec = C // 8
        eb0 = base0 // 8

        def issue_gathers(j, p, s):
            srow = src_v.at[s]
            pltpu.async_copy(als_hbm.at[srow], als_v[p], gsem[p])
            pltpu.async_copy(ald_hbm.at[dst_v.at[s]], ald_v[p], gsem[p])
            pltpu.async_copy(e_hbm.at[pl.ds(eb0 + j * ec, ec)], e_v[p],
                             gsem[p])
            pltpu.async_copy(h_hbm.at[srow], h_v[p], gsem[p])

        def wait_gathers(p):
            pltpu.make_async_copy(als_hbm.at[src_v.at[0]], als_v[p],
                                  gsem[p]).wait()
            pltpu.make_async_copy(ald_hbm.at[dst_v.at[0]], ald_v[p],
                                  gsem[p]).wait()
            pltpu.make_async_copy(e_hbm.at[pl.ds(eb0, ec)], e_v[p],
                                  gsem[p]).wait()
            pltpu.make_async_copy(h_hbm.at[src_v.at[0]], h_v[p],
                                  gsem[p]).wait()

        kidx = [jnp.full((DH,), k, jnp.int32) for k in range(H)]

        def compute(p):
            def edge_body(r, carry2):
                for jj in range(8):
                    c = r * 8 + jj
                    lg = (als_v[p][c] + ald_v[p][c]
                          + e_v[p][r, pl.ds(jj * DH, DH)])
                    lg = jnp.where(lg >= 0.0, lg, 0.2 * lg)
                    exv = jnp.exp(lg)
                    ex_v[p][c] = exv
                    for k in range(H):
                        hk = h_v[p][c, pl.ds(k * DH, DH)]
                        mk = exv.at[kidx[k]].get(mode="promise_in_bounds")
                        h_v[p][c, pl.ds(k * DH, DH)] = hk * mk
                return carry2

            lax.fori_loop(0, C // 8, edge_body, 0, unroll=2)

        def issue_scatters(p, s):
            drow = dst_v.at[s]
            pltpu.async_copy(ex_v[p], den_sh.at[drow], ssem[p], add=True)
            pltpu.async_copy(h_v[p], num_sh.at[drow], ssem[p], add=True)

        def wait_scatters(p):
            pltpu.make_async_copy(ex_v[p], den_sh.at[dst_v.at[0]],
                                  ssem[p]).wait()
            pltpu.make_async_copy(h_v[p], num_sh.at[dst_v.at[0]],
                                  ssem[p]).wait()

        wait_idx(0)
        issue_gathers(0, 0, 0)
        wait_idx(1)
        issue_gathers(1, 1, 1)
        wait_gathers(0)
        compute(0)
        issue_scatters(0, 0)

        def halfstep(j, pj, s_wait, s_issue, last_idx):
            wait_scatters((pj + 1) % 2)
            wait_idx(s_wait)
            issue_gathers(j + 1, (pj + 1) % 2, s_wait)
            if not last_idx:
                issue_idx(j + 3, s_issue)
            wait_gathers(pj)
            compute(pj)
            issue_scatters(pj, (s_wait - 1) % 4)

        halfstep(1, 1, 2, 0, False)

        def quad_body(u, carry):
            j = 4 * u + 2
            halfstep(j, 0, 3, 1, False)
            halfstep(j + 1, 1, 0, 2, False)
            halfstep(j + 2, 0, 1, 3, False)
            halfstep(j + 3, 1, 2, 0, False)
            return carry

        nquad = (nchunk - 5) // 4
        lax.fori_loop(0, nquad, quad_body, 0)

        halfstep(nchunk - 3, 0, 3, None, True)
        halfstep(nchunk - 2, 1, 0, None, True)
        wait_scatters(1)
        wait_gathers(0)
        compute(0)
        issue_scatters(0, 0)
        wait_scatters(0)

        plsc.subcore_barrier()
        pltpu.sync_copy(num_sh.at[pl.ds(zrow, rt)],
                        num_out.at[cid, pl.ds(zrow, rt)])
        pltpu.sync_copy(den_sh.at[pl.ds(zrow, rt)],
                        den_out.at[cid, pl.ds(zrow, rt)])

        @pl.when(sid == NS - 1)
        def _write_tail():
            tr = NS * rt
            pltpu.sync_copy(num_sh.at[pl.ds(tr, rem)],
                            num_out.at[cid, pl.ds(tr, rem)])
            pltpu.sync_copy(den_sh.at[pl.ds(tr, rem)],
                            den_out.at[cid, pl.ds(tr, rem)])

    return sc_kernel(src, dst, h, als, ald, e, zd, zh)


def _attn_mat(a):
    rows = jnp.arange(D)
    mask = (rows[:, None] // DH) == jnp.arange(HP)[None, :]
    return mask.astype(jnp.float32) * a.reshape(D)[:, None]


def kernel(x, size, edge_index, edge_attr, W0, a_src0, a_dst0, We0,
           W1, a_src1, a_dst1, We1):
    del size
    n = x.shape[0]
    src = edge_index[0]
    dst = edge_index[1]

    as0 = _attn_mat(a_src0)
    ad0 = _attn_mat(a_dst0)
    as1 = _attn_mat(a_src1)
    ad1 = _attn_mat(a_dst1)
    rows = jnp.arange(D)
    expm = ((rows[None, :] // DH) == jnp.arange(HP)[:, None]).astype(jnp.float32)
    zpad = jnp.zeros((We0.shape[0], HP - H), jnp.float32)
    we0p = jnp.concatenate([We0, zpad], axis=1)
    we1p = jnp.concatenate([We1, zpad], axis=1)
    eye8 = jnp.eye(8, dtype=jnp.float32)
    we0k = jnp.kron(eye8, we0p)
    we1k = jnp.kron(eye8, we1p)
    ea2 = edge_attr.reshape(-1, 8 * edge_attr.shape[1])
    zd = jnp.zeros((n, D), jnp.float32)
    zh = jnp.zeros((n, HP), jnp.float32)

    bn = 2000
    h0, als0, ald0 = _tc_pre(x, W0, as0, ad0, bn)
    e0 = _tc_edge_proj(ea2, we0k, 1000)
    e1 = _tc_edge_proj(ea2, we1k, 1000)
    num0, den0 = _sc_edges(src, dst, h0, als0, ald0, e0, zd, zh)
    h1, als1, ald1 = _tc_mid(num0, den0, expm, W1, as1, ad1, bn, n)
    num1, den1 = _sc_edges(src, dst, h1, als1, ald1, e1, zd, zh)
    return _tc_post(num1, den1, expm, bn, n)
